# initial kernel scaffold (unmeasured)
import jax
import jax.numpy as jnp
from jax import lax
from jax.experimental import pallas as pl
from jax.experimental.pallas import tpu as pltpu


def kernel(Q, K, V):
    b, s, h, d = Q.shape
    scale = d ** -0.5

    Qh = jnp.transpose(Q[0], (1, 0, 2))
    Kh = jnp.transpose(K[0], (1, 0, 2))
    Vh = jnp.transpose(V[0], (1, 0, 2))

    def body(q_ref, k_ref, v_ref, o_ref, ck_ref, cv_ref, send_sems, recv_sems):
        my_x = lax.axis_index("x")
        my_y = lax.axis_index("y")
        my_z = lax.axis_index("z")
        peer = (my_x, 1 - my_y, my_z)

        barrier_sem = pltpu.get_barrier_semaphore()
        pl.semaphore_signal(
            barrier_sem, inc=1, device_id=peer,
            device_id_type=pl.DeviceIdType.MESH,
        )
        pl.semaphore_wait(barrier_sem, 1)

        rk = pltpu.make_async_remote_copy(
            src_ref=k_ref, dst_ref=ck_ref,
            send_sem=send_sems.at[0], recv_sem=recv_sems.at[0],
            device_id=peer, device_id_type=pl.DeviceIdType.MESH,
        )
        rv = pltpu.make_async_remote_copy(
            src_ref=v_ref, dst_ref=cv_ref,
            send_sem=send_sems.at[1], recv_sem=recv_sems.at[1],
            device_id=peer, device_id_type=pl.DeviceIdType.MESH,
        )
        rk.start()
        rv.start()
        rk.wait()
        rv.wait()

        for head in range(h):
            q = q_ref[head]
            s1 = lax.dot_general(
                q, k_ref[head], (((1,), (1,)), ((), ())),
                preferred_element_type=jnp.float32,
            ) * scale
            s2 = lax.dot_general(
                q, ck_ref[head], (((1,), (1,)), ((), ())),
                preferred_element_type=jnp.float32,
            ) * scale
            st = jnp.concatenate([s1, s2], axis=1)
            m = jnp.max(st, axis=1, keepdims=True)
            p = jnp.exp(st - m)
            l = jnp.sum(p, axis=1, keepdims=True)
            o1 = lax.dot_general(
                p[:, :s], v_ref[head], (((1,), (0,)), ((), ())),
                preferred_element_type=jnp.float32,
            )
            o2 = lax.dot_general(
                p[:, s:], cv_ref[head], (((1,), (0,)), ((), ())),
                preferred_element_type=jnp.float32,
            )
            o_ref[head] = (o1 + o2) / l

    out = pl.pallas_call(
        body,
        out_shape=jax.ShapeDtypeStruct((h, s, d), jnp.float32),
        in_specs=[pl.BlockSpec(memory_space=pltpu.VMEM)] * 3,
        out_specs=pl.BlockSpec(memory_space=pltpu.VMEM),
        scratch_shapes=[
            pltpu.VMEM((h, s, d), jnp.float32),
            pltpu.VMEM((h, s, d), jnp.float32),
            pltpu.SemaphoreType.DMA((2,)),
            pltpu.SemaphoreType.DMA((2,)),
        ],
        compiler_params=pltpu.CompilerParams(collective_id=0),
    )(Qh, Kh, Vh)

    return jnp.transpose(out, (1, 0, 2))[None]


# baseline (device time: 333555 ns/iter reference)
import jax
import jax.numpy as jnp
from jax import lax
from jax.experimental import pallas as pl
from jax.experimental.pallas import tpu as pltpu


def kernel(Q, K, V):
    b, s, h, d = Q.shape
    scale = d ** -0.5

    Qh = jnp.transpose(Q[0], (1, 0, 2))
    Kh = jnp.transpose(K[0], (1, 0, 2))
    Vh = jnp.transpose(V[0], (1, 0, 2))

    def body(q_ref, k_ref, v_ref, o_ref, ck_ref, cv_ref, send_sems, recv_sems):
        my_x = lax.axis_index("x")
        my_y = lax.axis_index("y")
        my_z = lax.axis_index("z")
        peer = (my_x, 1 - my_y, my_z)

        barrier_sem = pltpu.get_barrier_semaphore()
        pl.semaphore_signal(
            barrier_sem, inc=1, device_id=peer,
            device_id_type=pl.DeviceIdType.MESH,
        )
        pl.semaphore_wait(barrier_sem, 1)

        rk = pltpu.make_async_remote_copy(
            src_ref=k_ref, dst_ref=ck_ref,
            send_sem=send_sems.at[0], recv_sem=recv_sems.at[0],
            device_id=peer, device_id_type=pl.DeviceIdType.MESH,
        )
        rv = pltpu.make_async_remote_copy(
            src_ref=v_ref, dst_ref=cv_ref,
            send_sem=send_sems.at[1], recv_sem=recv_sems.at[1],
            device_id=peer, device_id_type=pl.DeviceIdType.MESH,
        )
        rk.start()
        rv.start()
        rk.wait()
        rv.wait()

        q_chunk = 256
        n_chunks = s // q_chunk

        def step(i, _):
            head = i // n_chunks
            qc = i % n_chunks
            q = q_ref[head, pl.ds(qc * q_chunk, q_chunk), :]
            s1 = lax.dot_general(
                q, k_ref[head], (((1,), (1,)), ((), ())),
                preferred_element_type=jnp.float32,
            ) * scale
            s2 = lax.dot_general(
                q, ck_ref[head], (((1,), (1,)), ((), ())),
                preferred_element_type=jnp.float32,
            ) * scale
            m = jnp.maximum(
                jnp.max(s1, axis=1, keepdims=True),
                jnp.max(s2, axis=1, keepdims=True),
            )
            p1 = jnp.exp(s1 - m)
            p2 = jnp.exp(s2 - m)
            l = (
                jnp.sum(p1, axis=1, keepdims=True)
                + jnp.sum(p2, axis=1, keepdims=True)
            )
            o1 = lax.dot_general(
                p1, v_ref[head], (((1,), (0,)), ((), ())),
                preferred_element_type=jnp.float32,
            )
            o2 = lax.dot_general(
                p2, cv_ref[head], (((1,), (0,)), ((), ())),
                preferred_element_type=jnp.float32,
            )
            o_ref[head, pl.ds(qc * q_chunk, q_chunk), :] = (o1 + o2) / l
            return 0

        lax.fori_loop(0, h * n_chunks, step, 0)

    out = pl.pallas_call(
        body,
        out_shape=jax.ShapeDtypeStruct((h, s, d), jnp.float32),
        in_specs=[pl.BlockSpec(memory_space=pltpu.VMEM)] * 3,
        out_specs=pl.BlockSpec(memory_space=pltpu.VMEM),
        scratch_shapes=[
            pltpu.VMEM((h, s, d), jnp.float32),
            pltpu.VMEM((h, s, d), jnp.float32),
            pltpu.SemaphoreType.DMA((2,)),
            pltpu.SemaphoreType.DMA((2,)),
        ],
        compiler_params=pltpu.CompilerParams(
            collective_id=0,
            vmem_limit_bytes=60 * 1024 * 1024,
        ),
    )(Qh, Kh, Vh)

    return jnp.transpose(out, (1, 0, 2))[None]


# device time: 179498 ns/iter; 1.8583x vs baseline; 1.8583x over previous
import jax
import jax.numpy as jnp
from jax import lax
from jax.experimental import pallas as pl
from jax.experimental.pallas import tpu as pltpu

Q_CHUNK = 256


def kernel(Q, K, V):
    b, s, h, d = Q.shape
    scale = d ** -0.5
    n_chunks = s // Q_CHUNK

    Qh = jnp.transpose(Q[0].astype(jnp.bfloat16), (1, 0, 2))
    Kh = jnp.transpose(K[0].astype(jnp.bfloat16), (1, 0, 2))
    Vh = jnp.transpose(V[0].astype(jnp.bfloat16), (1, 0, 2))

    def body(q_ref, k_ref, v_ref, o_ref,
             ck_ref, cv_ref, m_scr, l_scr, send_sems, recv_sems):
        my_x = lax.axis_index("x")
        my_y = lax.axis_index("y")
        my_z = lax.axis_index("z")
        peer = (my_x, 1 - my_y, my_z)

        barrier_sem = pltpu.get_barrier_semaphore()
        pl.semaphore_signal(
            barrier_sem, inc=1, device_id=peer,
            device_id_type=pl.DeviceIdType.MESH,
        )
        pl.semaphore_wait(barrier_sem, 1)

        rk = pltpu.make_async_remote_copy(
            src_ref=k_ref, dst_ref=ck_ref,
            send_sem=send_sems.at[0], recv_sem=recv_sems.at[0],
            device_id=peer, device_id_type=pl.DeviceIdType.MESH,
        )
        rv = pltpu.make_async_remote_copy(
            src_ref=v_ref, dst_ref=cv_ref,
            send_sem=send_sems.at[1], recv_sem=recv_sems.at[1],
            device_id=peer, device_id_type=pl.DeviceIdType.MESH,
        )
        rk.start()
        rv.start()

        def phase1(i, _):
            head = i // n_chunks
            qc = i % n_chunks
            q = q_ref[head, pl.ds(qc * Q_CHUNK, Q_CHUNK), :]
            s1 = lax.dot_general(
                q, k_ref[head], (((1,), (1,)), ((), ())),
                preferred_element_type=jnp.float32,
            ) * scale
            m1 = jnp.max(s1, axis=1, keepdims=True)
            p1 = jnp.exp(s1 - m1)
            l1 = jnp.sum(p1, axis=1, keepdims=True)
            o1 = lax.dot_general(
                p1.astype(jnp.bfloat16), v_ref[head], (((1,), (0,)), ((), ())),
                preferred_element_type=jnp.float32,
            )
            o_ref[head, pl.ds(qc * Q_CHUNK, Q_CHUNK), :] = o1
            m_scr[i, :] = m1[:, 0]
            l_scr[i, :] = l1[:, 0]
            return 0

        lax.fori_loop(0, h * n_chunks, phase1, 0)

        rk.wait_recv()
        rv.wait_recv()
        rk.wait_send()
        rv.wait_send()

        def phase2(i, _):
            head = i // n_chunks
            qc = i % n_chunks
            q = q_ref[head, pl.ds(qc * Q_CHUNK, Q_CHUNK), :]
            s2 = lax.dot_general(
                q, ck_ref[head], (((1,), (1,)), ((), ())),
                preferred_element_type=jnp.float32,
            ) * scale
            m1 = m_scr[i, :][:, None]
            l1 = l_scr[i, :][:, None]
            m2 = jnp.max(s2, axis=1, keepdims=True)
            m = jnp.maximum(m1, m2)
            p2 = jnp.exp(s2 - m)
            l2 = jnp.sum(p2, axis=1, keepdims=True)
            o2 = lax.dot_general(
                p2.astype(jnp.bfloat16), cv_ref[head], (((1,), (0,)), ((), ())),
                preferred_element_type=jnp.float32,
            )
            alpha = jnp.exp(m1 - m)
            l = l1 * alpha + l2
            o1 = o_ref[head, pl.ds(qc * Q_CHUNK, Q_CHUNK), :]
            o_ref[head, pl.ds(qc * Q_CHUNK, Q_CHUNK), :] = (
                o1 * alpha + o2
            ) / l
            return 0

        lax.fori_loop(0, h * n_chunks, phase2, 0)

    out = pl.pallas_call(
        body,
        out_shape=jax.ShapeDtypeStruct((h, s, d), jnp.float32),
        in_specs=[pl.BlockSpec(memory_space=pltpu.VMEM)] * 3,
        out_specs=pl.BlockSpec(memory_space=pltpu.VMEM),
        scratch_shapes=[
            pltpu.VMEM((h, s, d), jnp.bfloat16),
            pltpu.VMEM((h, s, d), jnp.bfloat16),
            pltpu.VMEM((h * (s // Q_CHUNK), Q_CHUNK), jnp.float32),
            pltpu.VMEM((h * (s // Q_CHUNK), Q_CHUNK), jnp.float32),
            pltpu.SemaphoreType.DMA((2,)),
            pltpu.SemaphoreType.DMA((2,)),
        ],
        compiler_params=pltpu.CompilerParams(
            collective_id=0,
            vmem_limit_bytes=60 * 1024 * 1024,
        ),
    )(Qh, Kh, Vh)

    return jnp.transpose(out, (1, 0, 2))[None]


# device time: 131700 ns/iter; 2.5327x vs baseline; 1.3629x over previous
import math

import jax
import jax.numpy as jnp
from jax import lax
from jax.experimental import pallas as pl
from jax.experimental.pallas import tpu as pltpu

Q_CHUNK = 512


def kernel(Q, K, V):
    b, s, h, d = Q.shape
    scale = d ** -0.5
    n_chunks = s // Q_CHUNK

    Qh = jnp.transpose(
        (Q[0] * (scale * math.log2(math.e))).astype(jnp.bfloat16), (1, 0, 2)
    )
    Kh = jnp.transpose(K[0].astype(jnp.bfloat16), (1, 0, 2))
    Vh = jnp.transpose(V[0].astype(jnp.bfloat16), (1, 0, 2))

    def body(q_ref, k_ref, v_ref, o_ref,
             ck_ref, cv_ref, l_scr,
             send_k, send_v, recv_k, recv_v):
        my_x = lax.axis_index("x")
        my_y = lax.axis_index("y")
        my_z = lax.axis_index("z")
        peer = (my_x, 1 - my_y, my_z)

        def copy_head(hh, which):
            src, dst = (k_ref, ck_ref) if which == 0 else (v_ref, cv_ref)
            ss, rs = (send_k, recv_k) if which == 0 else (send_v, recv_v)
            return pltpu.make_async_remote_copy(
                src_ref=src.at[hh], dst_ref=dst.at[hh],
                send_sem=ss.at[hh], recv_sem=rs.at[hh],
                device_id=peer, device_id_type=pl.DeviceIdType.MESH,
            )

        barrier_sem = pltpu.get_barrier_semaphore()
        pl.semaphore_signal(
            barrier_sem, inc=1, device_id=peer,
            device_id_type=pl.DeviceIdType.MESH,
        )
        pl.semaphore_wait(barrier_sem, 1)

        for hh in range(h):
            copy_head(hh, 0).start()
            copy_head(hh, 1).start()

        def phase1(i, _):
            head = i // n_chunks
            qc = i % n_chunks
            q = q_ref[head, pl.ds(qc * Q_CHUNK, Q_CHUNK), :]
            s1 = lax.dot_general(
                q, k_ref[head], (((1,), (1,)), ((), ())),
                preferred_element_type=jnp.float32,
            )
            p1 = jnp.exp2(s1)
            l1 = jnp.sum(p1, axis=1, keepdims=True)
            o1 = lax.dot_general(
                p1.astype(jnp.bfloat16), v_ref[head], (((1,), (0,)), ((), ())),
                preferred_element_type=jnp.float32,
            )
            o_ref[head, pl.ds(qc * Q_CHUNK, Q_CHUNK), :] = o1
            l_scr[i, :] = l1[:, 0]
            return 0

        lax.fori_loop(0, h * n_chunks, phase1, 0)

        for head in range(h):
            copy_head(head, 0).wait_recv()
            copy_head(head, 1).wait_recv()
            for qc in range(n_chunks):
                i = head * n_chunks + qc
                q = q_ref[head, pl.ds(qc * Q_CHUNK, Q_CHUNK), :]
                s2 = lax.dot_general(
                    q, ck_ref[head], (((1,), (1,)), ((), ())),
                    preferred_element_type=jnp.float32,
                )
                p2 = jnp.exp2(s2)
                l2 = jnp.sum(p2, axis=1, keepdims=True)
                o2 = lax.dot_general(
                    p2.astype(jnp.bfloat16), cv_ref[head],
                    (((1,), (0,)), ((), ())),
                    preferred_element_type=jnp.float32,
                )
                l1 = l_scr[i, :][:, None]
                o1 = o_ref[head, pl.ds(qc * Q_CHUNK, Q_CHUNK), :]
                o_ref[head, pl.ds(qc * Q_CHUNK, Q_CHUNK), :] = (
                    (o1 + o2) / (l1 + l2)
                )

        for hh in range(h):
            copy_head(hh, 0).wait_send()
            copy_head(hh, 1).wait_send()

    out = pl.pallas_call(
        body,
        out_shape=jax.ShapeDtypeStruct((h, s, d), jnp.float32),
        in_specs=[pl.BlockSpec(memory_space=pltpu.VMEM)] * 3,
        out_specs=pl.BlockSpec(memory_space=pltpu.VMEM),
        scratch_shapes=[
            pltpu.VMEM((h, s, d), jnp.bfloat16),
            pltpu.VMEM((h, s, d), jnp.bfloat16),
            pltpu.VMEM((h * (s // Q_CHUNK), Q_CHUNK), jnp.float32),
            pltpu.SemaphoreType.DMA((16,)),
            pltpu.SemaphoreType.DMA((16,)),
            pltpu.SemaphoreType.DMA((16,)),
            pltpu.SemaphoreType.DMA((16,)),
        ],
        compiler_params=pltpu.CompilerParams(
            collective_id=0,
            vmem_limit_bytes=60 * 1024 * 1024,
        ),
    )(Qh, Kh, Vh)

    return jnp.transpose(out, (1, 0, 2))[None]


# device time: 104217 ns/iter; 3.2006x vs baseline; 1.2637x over previous
import math

import jax
import jax.numpy as jnp
from jax import lax
from jax.experimental import pallas as pl
from jax.experimental.pallas import tpu as pltpu

Q_CHUNK = 512


def kernel(Q, K, V):
    b, s, h, d = Q.shape
    n_chunks = s // Q_CHUNK
    q_const = (d ** -0.5) * math.log2(math.e)

    def body(q_hbm, k_hbm, v_hbm, o_hbm,
             kst, vst, qst, kb, vb, qb, ck, cv, ov, l_scr,
             kd_sem, vd_sem, qd_sem, od_sem,
             send_k, send_v, recv_k, recv_v):
        my_x = lax.axis_index("x")
        my_y = lax.axis_index("y")
        my_z = lax.axis_index("z")
        peer = (my_x, 1 - my_y, my_z)

        def in_dma(hbm, stage, sem, hh):
            return pltpu.make_async_copy(
                hbm.at[0, :, hh, :], stage.at[hh], sem.at[hh]
            )

        def rdma(hh, which):
            src, dst = (kb, ck) if which == 0 else (vb, cv)
            ss, rs = (send_k, recv_k) if which == 0 else (send_v, recv_v)
            return pltpu.make_async_remote_copy(
                src_ref=src.at[hh], dst_ref=dst.at[hh],
                send_sem=ss.at[hh], recv_sem=rs.at[hh],
                device_id=peer, device_id_type=pl.DeviceIdType.MESH,
            )

        barrier_sem = pltpu.get_barrier_semaphore()
        pl.semaphore_signal(
            barrier_sem, inc=1, device_id=peer,
            device_id_type=pl.DeviceIdType.MESH,
        )
        pl.semaphore_wait(barrier_sem, 1)

        for hh in range(h):
            in_dma(k_hbm, kst, kd_sem, hh).start()
            in_dma(v_hbm, vst, vd_sem, hh).start()
        for hh in range(h):
            in_dma(k_hbm, kst, kd_sem, hh).wait()
            kb[hh] = kst[hh].astype(jnp.bfloat16)
            rdma(hh, 0).start()
            in_dma(v_hbm, vst, vd_sem, hh).wait()
            vb[hh] = vst[hh].astype(jnp.bfloat16)
            rdma(hh, 1).start()

        for hh in range(h):
            in_dma(q_hbm, qst, qd_sem, hh).start()
        for hh in range(h):
            in_dma(q_hbm, qst, qd_sem, hh).wait()
            qb[hh] = (qst[hh] * q_const).astype(jnp.bfloat16)

        def phase1(i, _):
            head = i // n_chunks
            qc = i % n_chunks
            q = qb[head, pl.ds(qc * Q_CHUNK, Q_CHUNK), :]
            s1 = lax.dot_general(
                q, kb[head], (((1,), (1,)), ((), ())),
                preferred_element_type=jnp.float32,
            )
            p1 = jnp.exp2(s1)
            l1 = jnp.sum(p1, axis=1, keepdims=True)
            o1 = lax.dot_general(
                p1.astype(jnp.bfloat16), vb[head], (((1,), (0,)), ((), ())),
                preferred_element_type=jnp.float32,
            )
            ov[head, pl.ds(qc * Q_CHUNK, Q_CHUNK), :] = o1
            l_scr[i, :] = l1[:, 0]
            return 0

        lax.fori_loop(0, h * n_chunks, phase1, 0)

        for head in range(h):
            rdma(head, 0).wait_recv()
            rdma(head, 1).wait_recv()
            for qc in range(n_chunks):
                i = head * n_chunks + qc
                q = qb[head, pl.ds(qc * Q_CHUNK, Q_CHUNK), :]
                s2 = lax.dot_general(
                    q, ck[head], (((1,), (1,)), ((), ())),
                    preferred_element_type=jnp.float32,
                )
                p2 = jnp.exp2(s2)
                l2 = jnp.sum(p2, axis=1, keepdims=True)
                o2 = lax.dot_general(
                    p2.astype(jnp.bfloat16), cv[head], (((1,), (0,)), ((), ())),
                    preferred_element_type=jnp.float32,
                )
                l1 = l_scr[i, :][:, None]
                o1 = ov[head, pl.ds(qc * Q_CHUNK, Q_CHUNK), :]
                ov[head, pl.ds(qc * Q_CHUNK, Q_CHUNK), :] = (
                    (o1 + o2) / (l1 + l2)
                )
            pltpu.make_async_copy(
                ov.at[head], o_hbm.at[0, :, head, :], od_sem.at[head]
            ).start()

        for head in range(h):
            pltpu.make_async_copy(
                ov.at[head], o_hbm.at[0, :, head, :], od_sem.at[head]
            ).wait()
        for hh in range(h):
            rdma(hh, 0).wait_send()
            rdma(hh, 1).wait_send()

    out = pl.pallas_call(
        body,
        out_shape=jax.ShapeDtypeStruct((b, s, h, d), jnp.float32),
        in_specs=[pl.BlockSpec(memory_space=pl.ANY)] * 3,
        out_specs=pl.BlockSpec(memory_space=pl.ANY),
        scratch_shapes=[
            pltpu.VMEM((h, s, d), jnp.float32),
            pltpu.VMEM((h, s, d), jnp.float32),
            pltpu.VMEM((h, s, d), jnp.float32),
            pltpu.VMEM((h, s, d), jnp.bfloat16),
            pltpu.VMEM((h, s, d), jnp.bfloat16),
            pltpu.VMEM((h, s, d), jnp.bfloat16),
            pltpu.VMEM((h, s, d), jnp.bfloat16),
            pltpu.VMEM((h, s, d), jnp.bfloat16),
            pltpu.VMEM((h, s, d), jnp.float32),
            pltpu.VMEM((h * (s // Q_CHUNK), Q_CHUNK), jnp.float32),
            pltpu.SemaphoreType.DMA((16,)),
            pltpu.SemaphoreType.DMA((16,)),
            pltpu.SemaphoreType.DMA((16,)),
            pltpu.SemaphoreType.DMA((16,)),
            pltpu.SemaphoreType.DMA((16,)),
            pltpu.SemaphoreType.DMA((16,)),
            pltpu.SemaphoreType.DMA((16,)),
            pltpu.SemaphoreType.DMA((16,)),
        ],
        compiler_params=pltpu.CompilerParams(
            collective_id=0,
            vmem_limit_bytes=63 * 1024 * 1024,
        ),
    )(Q, K, V)

    return out
